# baseline (speedup 1.0000x reference)
import jax
import jax.numpy as jnp
from jax.experimental import pallas as pl
from jax.experimental.pallas import tpu as pltpu

_R = 512
_N = 2048
_BPW = _N // _R
_G = 3 * _BPW
_NBUF = 3


def _body(x_ref, w1_ref, w2_ref, w3_ref, b_ref, mw_ref, mb_ref,
          w1o_ref, w2o_ref, w3o_ref, out_ref,
          buf, h1_ref, h2_ref, in_sem, out_sem):
    g = pl.program_id(0)
    l = g // _BPW
    t = g % _BPW
    slot = jax.lax.rem(g, _NBUF)

    w_refs = (w1_ref, w2_ref, w3_ref)
    wo_refs = (w1o_ref, w2o_ref, w3o_ref)

    def in_copy(gi):
        li, ti, si = gi // _BPW, gi % _BPW, jax.lax.rem(gi, _NBUF)
        def op(k, method):
            @pl.when(li == k)
            def _():
                c = pltpu.make_async_copy(
                    w_refs[k].at[pl.ds(ti * _R, _R), :],
                    buf.at[si], in_sem.at[si])
                getattr(c, method)()
        return op

    def out_copy(gi):
        li, ti, si = gi // _BPW, gi % _BPW, jax.lax.rem(gi, _NBUF)
        def op(k, method):
            @pl.when(li == k)
            def _():
                c = pltpu.make_async_copy(
                    buf.at[si], wo_refs[k].at[pl.ds(ti * _R, _R), :],
                    out_sem.at[si])
                getattr(c, method)()
        return op

    def start_in(gi):
        op = in_copy(gi)
        op(0, "start"); op(1, "start"); op(2, "start")

    def wait_in(gi):
        op = in_copy(gi)
        op(0, "wait"); op(1, "wait"); op(2, "wait")

    def start_out(gi):
        op = out_copy(gi)
        op(0, "start"); op(1, "start"); op(2, "start")

    def wait_out(gi):
        op = out_copy(gi)
        op(0, "wait"); op(1, "wait"); op(2, "wait")

    @pl.when(g == 0)
    def _():
        start_in(0)

    wait_in(g)

    @pl.when(g + 1 < _G)
    def _():
        @pl.when(g >= _NBUF - 1)
        def _():
            wait_out(g - (_NBUF - 1))
        start_in(g + 1)

    vec = jnp.where(l == 0, x_ref[...],
                    jnp.where(l == 1, h1_ref[...], h2_ref[...]))
    y = jax.lax.dot_general(
        vec, buf[slot], (((1,), (1,)), ((), ())),
        preferred_element_type=jnp.float32,
    )
    h = jnp.maximum(y + b_ref[pl.ds(l, 1), pl.ds(t * _R, _R)], 0.0)

    @pl.when(l == 0)
    def _():
        h1_ref[0:1, pl.ds(t * _R, _R)] = h

    @pl.when(l == 1)
    def _():
        h2_ref[0:1, pl.ds(t * _R, _R)] = h

    @pl.when(l == 2)
    def _():
        out_ref[...] = h

    @pl.when(t == 0)
    def _():
        cols_h = jax.lax.broadcasted_iota(jnp.int32, h.shape, 1)
        h0 = jnp.sum(jnp.where(cols_h == 0, h, 0.0))
        row0 = buf[slot, 0:1, :]
        cols_w = jax.lax.broadcasted_iota(jnp.int32, row0.shape, 1)
        w00 = jnp.sum(jnp.where(cols_w == 0, row0, 0.0))
        s = jnp.sum(jnp.where(cols_w == 0, vec, 0.0))
        new00 = (s * mw_ref[0, 0] + w00 * mw_ref[0, 1]
                 + h0 * mw_ref[0, 2] + mb_ref[0])
        buf[slot, 0:1, :] = jnp.where(cols_w == 0, new00, row0)

    start_out(g)

    @pl.when(g == _G - 1)
    def _():
        for d in range(_NBUF - 1, -1, -1):
            wait_out(g - d)


def kernel(x, W1, b1, W2, b2, W3, b3, meta_W, meta_b):
    b_all = jnp.concatenate(
        [b1.reshape(1, -1), b2.reshape(1, -1), b3.reshape(1, -1)], axis=0)
    hbm = pl.BlockSpec(memory_space=pl.ANY)
    W1n, W2n, W3n, out = pl.pallas_call(
        _body,
        grid=(_G,),
        in_specs=[
            pl.BlockSpec((1, _N), lambda g: (0, 0)),
            hbm, hbm, hbm,
            pl.BlockSpec((3, _N), lambda g: (0, 0)),
            pl.BlockSpec(memory_space=pltpu.SMEM),
            pl.BlockSpec(memory_space=pltpu.SMEM),
        ],
        out_specs=[
            hbm, hbm, hbm,
            pl.BlockSpec((1, _R), lambda g: (0, jnp.clip(g - 2 * _BPW, 0, _BPW - 1))),
        ],
        out_shape=[
            jax.ShapeDtypeStruct((_N, _N), jnp.float32),
            jax.ShapeDtypeStruct((_N, _N), jnp.float32),
            jax.ShapeDtypeStruct((_N, _N), jnp.float32),
            jax.ShapeDtypeStruct((1, _N), jnp.float32),
        ],
        scratch_shapes=[
            pltpu.VMEM((_NBUF, _R, _N), jnp.float32),
            pltpu.VMEM((1, _N), jnp.float32),
            pltpu.VMEM((1, _N), jnp.float32),
            pltpu.SemaphoreType.DMA((_NBUF,)),
            pltpu.SemaphoreType.DMA((_NBUF,)),
        ],
    )(x, W1, W2, W3, b_all, meta_W, meta_b)
    return (out, W1n, W2n, W3n)
